# baseline (speedup 1.0000x reference)
import functools

import jax
import jax.numpy as jnp
from jax import lax
from jax.experimental import pallas as pl
from jax.experimental.pallas import tpu as pltpu
from jax.experimental.pallas import tpu_sc as plsc

N_NODES = 10000
N_EDGES = 320000
DIM = 128
HALF = 64
N_GRAPHS = 64

NCORES = 2
NSUB = 16
EPT = N_EDGES // NSUB
CHUNK = 125
NSEG = 2
SEG = EPT // (NSEG * CHUNK)
WCH = 80
ROWS_PT = 624
TAIL_ROWS = N_NODES - NSUB * ROWS_PT
CNT_W = 16


def _fill_const(buf, rows, width, value):
    def row(i, _):
        for j in range(width // 16):
            buf[i, pl.ds(j * 16, 16)] = jnp.full((16,), value, jnp.float32)
        return 0
    lax.fori_loop(0, rows, row, 0)


def _zero_slice(zbuf, sh, base, is_last, sem):
    n_full = ROWS_PT // WCH
    rem = ROWS_PT - n_full * WCH
    for k in range(n_full):
        pltpu.async_copy(zbuf, sh.at[pl.ds(base + k * WCH, WCH)], sem)
    pltpu.async_copy(zbuf.at[pl.ds(0, rem)],
                     sh.at[pl.ds(base + n_full * WCH, rem)], sem)

    @pl.when(is_last)
    def _():
        pltpu.async_copy(zbuf.at[pl.ds(0, TAIL_ROWS)],
                         sh.at[pl.ds(NSUB * ROWS_PT, TAIL_ROWS)], sem)
    for k in range(n_full):
        pltpu.make_async_copy(zbuf, sh.at[pl.ds(base + k * WCH, WCH)],
                              sem).wait()
    pltpu.make_async_copy(zbuf.at[pl.ds(0, rem)],
                          sh.at[pl.ds(base + n_full * WCH, rem)], sem).wait()

    @pl.when(is_last)
    def _():
        pltpu.make_async_copy(zbuf.at[pl.ds(0, TAIL_ROWS)],
                              sh.at[pl.ds(NSUB * ROWS_PT, TAIL_ROWS)],
                              sem).wait()


def _write_cnt(sh, out_ref, stage, base, is_last):
    n_full = ROWS_PT // WCH
    rem = ROWS_PT - n_full * WCH
    for k in range(n_full):
        pltpu.sync_copy(sh.at[pl.ds(base + k * WCH, WCH)], stage)
        pltpu.sync_copy(stage, out_ref.at[pl.ds(base + k * WCH, WCH)])
    if rem:
        pltpu.sync_copy(sh.at[pl.ds(base + n_full * WCH, rem)],
                        stage.at[pl.ds(0, rem)])
        pltpu.sync_copy(stage.at[pl.ds(0, rem)],
                        out_ref.at[pl.ds(base + n_full * WCH, rem)])

    @pl.when(is_last)
    def _():
        pltpu.sync_copy(sh.at[pl.ds(NSUB * ROWS_PT, TAIL_ROWS)],
                        stage.at[pl.ds(0, TAIL_ROWS)])
        pltpu.sync_copy(stage.at[pl.ds(0, TAIL_ROWS)],
                        out_ref.at[pl.ds(NSUB * ROWS_PT, TAIL_ROWS)])


def _write_slice(sh, out_ref, stages, wsem0, wsem1, base, is_last):
    wsems = (wsem0, wsem1)
    n_full = ROWS_PT // WCH
    rem = ROWS_PT - n_full * WCH
    for k in range(n_full):
        st, sem = stages[k % 2], wsems[k % 2]
        if k >= 2:
            pltpu.make_async_copy(
                st, out_ref.at[pl.ds(base + (k - 2) * WCH, WCH)], sem).wait()
        pltpu.sync_copy(sh.at[pl.ds(base + k * WCH, WCH)], st)
        pltpu.async_copy(st, out_ref.at[pl.ds(base + k * WCH, WCH)], sem)
    for k in (n_full - 2, n_full - 1):
        st, sem = stages[k % 2], wsems[k % 2]
        pltpu.make_async_copy(
            st, out_ref.at[pl.ds(base + k * WCH, WCH)], sem).wait()
    if rem:
        st = stages[0]
        pltpu.sync_copy(sh.at[pl.ds(base + n_full * WCH, rem)],
                        st.at[pl.ds(0, rem)])
        pltpu.sync_copy(st.at[pl.ds(0, rem)],
                        out_ref.at[pl.ds(base + n_full * WCH, rem)])

    @pl.when(is_last)
    def _():
        st = stages[1]
        pltpu.sync_copy(sh.at[pl.ds(NSUB * ROWS_PT, TAIL_ROWS)],
                        st.at[pl.ds(0, TAIL_ROWS)])
        pltpu.sync_copy(st.at[pl.ds(0, TAIL_ROWS)],
                        out_ref.at[pl.ds(NSUB * ROWS_PT, TAIL_ROWS)])


NBUF = 4


def _sc_agg_body(with_cnt, hst_hbm, srcr_hbm, dstr_hbm, *rest):
    if with_cnt:
        (part_hbm, cntp_hbm, src_v, dst_v, r0, r1, r2, r3, stage, stage2,
         zbuf, ones_v, czbuf, s0, s1, s2, s3, w0, w1, w2,
         csem, isem, agg_sh, cnt_sh) = rest
    else:
        (part_hbm, src_v, dst_v, r0, r1, r2, r3, stage, stage2, zbuf,
         s0, s1, s2, s3, w0, w1, w2, csem, isem, agg_sh) = rest
    rows = (r0, r1, r2, r3)
    sems = (s0, s1, s2, s3)

    c = lax.axis_index("c")
    s = lax.axis_index("s")
    base = s * ROWS_PT
    is_last = s == NSUB - 1
    h_hbm = hst_hbm.at[c]

    pltpu.async_copy(srcr_hbm.at[s, 0], src_v, isem)
    pltpu.async_copy(dstr_hbm.at[s, 0], dst_v, isem)

    _fill_const(zbuf, WCH, HALF, 0.0)
    if with_cnt:
        _fill_const(czbuf, WCH, CNT_W, 0.0)
        _fill_const(ones_v, CHUNK, CNT_W, 1.0)

    _zero_slice(zbuf, agg_sh, base, is_last, csem)
    if with_cnt:
        _zero_slice(czbuf, cnt_sh, base, is_last, w2)

    pltpu.make_async_copy(srcr_hbm.at[s, 0], src_v, isem).wait()
    pltpu.make_async_copy(dstr_hbm.at[s, 0], dst_v, isem).wait()
    for b in range(NBUF):
        pltpu.async_copy(h_hbm.at[src_v.at[b]], rows[b], sems[b])
    plsc.subcore_barrier()

    for seg in range(NSEG):
        def group(i, _):
            j0 = NBUF * i
            for b in range(NBUF):
                j = j0 + b
                pltpu.make_async_copy(h_hbm.at[src_v.at[j]], rows[b],
                                      sems[b]).wait()
                pltpu.sync_copy(rows[b], agg_sh.at[dst_v.at[j]], add=True)
                if with_cnt:
                    @pl.when(c == seg)
                    def _():
                        pltpu.sync_copy(ones_v, cnt_sh.at[dst_v.at[j]],
                                        add=True)

                @pl.when(j + NBUF < SEG)
                def _():
                    pltpu.async_copy(h_hbm.at[src_v.at[j + NBUF]], rows[b],
                                     sems[b])
            return 0
        lax.fori_loop(0, SEG // NBUF, group, 0)

        if seg + 1 < NSEG:
            pltpu.async_copy(srcr_hbm.at[s, seg + 1], src_v, isem)
            pltpu.async_copy(dstr_hbm.at[s, seg + 1], dst_v, isem)
            pltpu.make_async_copy(srcr_hbm.at[s, seg + 1], src_v, isem).wait()
            pltpu.make_async_copy(dstr_hbm.at[s, seg + 1], dst_v, isem).wait()
            for b in range(NBUF):
                pltpu.async_copy(h_hbm.at[src_v.at[b]], rows[b], sems[b])

    plsc.subcore_barrier()

    _write_slice(agg_sh, part_hbm.at[c], (stage, stage2), w0, w1,
                 base, is_last)
    if with_cnt:
        _write_cnt(cnt_sh, cntp_hbm.at[c], czbuf, base, is_last)


def _make_sc_agg(with_cnt):
    mesh = plsc.VectorSubcoreMesh(core_axis_name="c", subcore_axis_name="s")
    out_type = [
        jax.ShapeDtypeStruct((NCORES, N_NODES, HALF), jnp.float32)]
    scratch = [
        pltpu.VMEM((SEG, CHUNK), jnp.int32),
        pltpu.VMEM((SEG, CHUNK), jnp.int32),
        pltpu.VMEM((CHUNK, HALF), jnp.float32),
        pltpu.VMEM((CHUNK, HALF), jnp.float32),
        pltpu.VMEM((CHUNK, HALF), jnp.float32),
        pltpu.VMEM((CHUNK, HALF), jnp.float32),
        pltpu.VMEM((WCH, HALF), jnp.float32),
        pltpu.VMEM((WCH, HALF), jnp.float32),
        pltpu.VMEM((WCH, HALF), jnp.float32),
        pltpu.SemaphoreType.DMA,
        pltpu.SemaphoreType.DMA,
        pltpu.SemaphoreType.DMA,
        pltpu.SemaphoreType.DMA,
        pltpu.SemaphoreType.DMA,
        pltpu.SemaphoreType.DMA,
        pltpu.SemaphoreType.DMA,
        pltpu.SemaphoreType.DMA,
        pltpu.SemaphoreType.DMA,
        pltpu.VMEM_SHARED((N_NODES, HALF), jnp.float32),
    ]
    if with_cnt:
        out_type.append(
            jax.ShapeDtypeStruct((NCORES, N_NODES, CNT_W), jnp.float32))
        scratch = scratch[:9] + [
            pltpu.VMEM((CHUNK, CNT_W), jnp.float32),
            pltpu.VMEM((WCH, CNT_W), jnp.float32),
        ] + scratch[9:] + [
            pltpu.VMEM_SHARED((N_NODES, CNT_W), jnp.float32),
        ]
    return pl.kernel(
        functools.partial(_sc_agg_body, with_cnt),
        out_type=out_type,
        mesh=mesh,
        scratch_types=scratch,
        compiler_params=pltpu.CompilerParams(use_tc_tiling_on_sc=False),
    )


def _combine_body(first, hst_ref, part_ref, aux_ref, wl_ref, bl_ref,
                  wr_ref, o_ref, *maybe_invc_out):
    h = jnp.concatenate([hst_ref[0], hst_ref[1]], axis=1)
    agg = jnp.concatenate([part_ref[0], part_ref[1]], axis=1)
    if first:
        cnt = (aux_ref[0] + aux_ref[1])[:, 0:1]
        invc = 1.0 / jnp.maximum(cnt, 1.0)
        maybe_invc_out[0][...] = invc
    else:
        invc = aux_ref[...]
    agg = agg * invc
    out = (jnp.dot(agg, wl_ref[...], preferred_element_type=jnp.float32,
                     precision=lax.Precision.HIGHEST)
           + jnp.dot(h, wr_ref[...], preferred_element_type=jnp.float32,
                     precision=lax.Precision.HIGHEST)
           + bl_ref[...])
    out = jnp.maximum(out, 0.0)
    o_ref[0] = out[:, :HALF]
    o_ref[1] = out[:, HALF:]


def _combine_tc(first, h_st, part, aux, Wl, bl, Wr):
    blk = 2000
    grid = N_NODES // blk
    if first:
        aux_spec = pl.BlockSpec((NCORES, blk, CNT_W), lambda i: (0, i, 0))
    else:
        aux_spec = pl.BlockSpec((blk, 1), lambda i: (i, 0))
    out_shape = [jax.ShapeDtypeStruct((NCORES, N_NODES, HALF), jnp.float32)]
    out_specs = [pl.BlockSpec((NCORES, blk, HALF), lambda i: (0, i, 0))]
    if first:
        out_shape.append(jax.ShapeDtypeStruct((N_NODES, 1), jnp.float32))
        out_specs.append(pl.BlockSpec((blk, 1), lambda i: (i, 0)))
    return pl.pallas_call(
        functools.partial(_combine_body, first),
        grid=(grid,),
        in_specs=[
            pl.BlockSpec((NCORES, blk, HALF), lambda i: (0, i, 0)),
            pl.BlockSpec((NCORES, blk, HALF), lambda i: (0, i, 0)),
            aux_spec,
            pl.BlockSpec((DIM, DIM), lambda i: (0, 0)),
            pl.BlockSpec((1, DIM), lambda i: (0, 0)),
            pl.BlockSpec((DIM, DIM), lambda i: (0, 0)),
        ],
        out_specs=out_specs,
        out_shape=out_shape,
    )(h_st, part, aux, Wl, bl.reshape(1, DIM), Wr)


def _last_body(nblk, hst_ref, part_ref, invc_ref, wl_ref, bl_ref,
               wr_ref, batch_ref, wc1_ref, bc1_ref, wc2_ref, bc2_ref, out_ref,
               gsum, cntg):
    i = pl.program_id(0)
    blk = hst_ref.shape[1]

    @pl.when(i == 0)
    def _():
        gsum[...] = jnp.zeros_like(gsum)
        cntg[...] = jnp.zeros_like(cntg)

    h = jnp.concatenate([hst_ref[0], hst_ref[1]], axis=1)
    agg = jnp.concatenate([part_ref[0], part_ref[1]], axis=1)
    agg = agg * invc_ref[...]
    h3 = (jnp.dot(agg, wl_ref[...], preferred_element_type=jnp.float32,
                     precision=lax.Precision.HIGHEST)
          + jnp.dot(h, wr_ref[...], preferred_element_type=jnp.float32,
                     precision=lax.Precision.HIGHEST)
          + bl_ref[...])
    h3 = jnp.maximum(h3, 0.0)

    b = batch_ref[...]
    gid = lax.broadcasted_iota(jnp.int32, (blk, N_GRAPHS), 1)
    oh = (b == gid).astype(jnp.float32)
    gsum[...] += lax.dot_general(oh, h3, (((0,), (0,)), ((), ())),
                                 preferred_element_type=jnp.float32,
                                 precision=lax.Precision.HIGHEST)
    cntg[...] += jnp.sum(oh, axis=0)[:, None]

    @pl.when(i == nblk - 1)
    def _():
        g = gsum[...] / jnp.maximum(cntg[...], 1.0)
        z = jnp.maximum(
            jnp.dot(g, wc1_ref[...], preferred_element_type=jnp.float32,
                     precision=lax.Precision.HIGHEST)
            + bc1_ref[...], 0.0)
        out_ref[...] = (
            jnp.dot(z, wc2_ref[...], preferred_element_type=jnp.float32,
                     precision=lax.Precision.HIGHEST)
            + bc2_ref[...])


def _last_tc(h_st, part, invc, Wl, bl, Wr, batch, Wc1, bc1, Wc2, bc2):
    blk = 2000
    grid = N_NODES // blk
    return pl.pallas_call(
        functools.partial(_last_body, grid),
        grid=(grid,),
        in_specs=[
            pl.BlockSpec((NCORES, blk, HALF), lambda i: (0, i, 0)),
            pl.BlockSpec((NCORES, blk, HALF), lambda i: (0, i, 0)),
            pl.BlockSpec((blk, 1), lambda i: (i, 0)),
            pl.BlockSpec((DIM, DIM), lambda i: (0, 0)),
            pl.BlockSpec((1, DIM), lambda i: (0, 0)),
            pl.BlockSpec((DIM, DIM), lambda i: (0, 0)),
            pl.BlockSpec((blk, 1), lambda i: (i, 0)),
            pl.BlockSpec((DIM, DIM), lambda i: (0, 0)),
            pl.BlockSpec((1, DIM), lambda i: (0, 0)),
            pl.BlockSpec((DIM, 2), lambda i: (0, 0)),
            pl.BlockSpec((1, 2), lambda i: (0, 0)),
        ],
        out_specs=pl.BlockSpec((N_GRAPHS, 2), lambda i: (0, 0)),
        out_shape=jax.ShapeDtypeStruct((N_GRAPHS, 2), jnp.float32),
        scratch_shapes=[pltpu.VMEM((N_GRAPHS, DIM), jnp.float32),
                        pltpu.VMEM((N_GRAPHS, 1), jnp.float32)],
    )(h_st, part, invc, Wl, bl.reshape(1, DIM), Wr,
      batch.reshape(N_NODES, 1), Wc1, bc1.reshape(1, DIM), Wc2,
      bc2.reshape(1, 2))


def kernel(x, edge_index, batch, Wl0, bl0, Wr0, Wl1, bl1, Wr1, Wl2, bl2, Wr2,
           Wc1, bc1, Wc2, bc2):
    src = edge_index[0].reshape(NSUB, NSEG, SEG, CHUNK)
    dst = edge_index[1].reshape(NSUB, NSEG, SEG, CHUNK)
    x_st = jnp.stack([x[:, :HALF], x[:, HALF:]])

    agg_cnt = _make_sc_agg(True)
    agg = _make_sc_agg(False)

    part0, cntp = agg_cnt(x_st, src, dst)
    h1, invc = _combine_tc(True, x_st, part0, cntp, Wl0, bl0, Wr0)
    part1 = agg(h1, src, dst)[0]
    h2 = _combine_tc(False, h1, part1, invc, Wl1, bl1, Wr1)[0]
    part2 = agg(h2, src, dst)[0]
    return _last_tc(h2, part2, invc, Wl2, bl2, Wr2, batch,
                    Wc1, bc1, Wc2, bc2)


# speedup vs baseline: 1.0281x; 1.0281x over previous
import functools

import jax
import jax.numpy as jnp
from jax import lax
from jax.experimental import pallas as pl
from jax.experimental.pallas import tpu as pltpu
from jax.experimental.pallas import tpu_sc as plsc

N_NODES = 10000
N_EDGES = 320000
DIM = 128
HALF = 64
N_GRAPHS = 64

NCORES = 2
NSUB = 16
EPT = N_EDGES // NSUB
CHUNK = 125
NSEG = 2
SEG = EPT // (NSEG * CHUNK)
WCH = 80
ROWS_PT = 624
TAIL_ROWS = N_NODES - NSUB * ROWS_PT
CNT_W = 16


def _fill_const(buf, rows, width, value):
    def row(i, _):
        for j in range(width // 16):
            buf[i, pl.ds(j * 16, 16)] = jnp.full((16,), value, jnp.float32)
        return 0
    lax.fori_loop(0, rows, row, 0)


def _zero_slice(zbuf, sh, base, is_last, sem):
    n_full = ROWS_PT // WCH
    rem = ROWS_PT - n_full * WCH
    for k in range(n_full):
        pltpu.async_copy(zbuf, sh.at[pl.ds(base + k * WCH, WCH)], sem)
    pltpu.async_copy(zbuf.at[pl.ds(0, rem)],
                     sh.at[pl.ds(base + n_full * WCH, rem)], sem)

    @pl.when(is_last)
    def _():
        pltpu.async_copy(zbuf.at[pl.ds(0, TAIL_ROWS)],
                         sh.at[pl.ds(NSUB * ROWS_PT, TAIL_ROWS)], sem)
    for k in range(n_full):
        pltpu.make_async_copy(zbuf, sh.at[pl.ds(base + k * WCH, WCH)],
                              sem).wait()
    pltpu.make_async_copy(zbuf.at[pl.ds(0, rem)],
                          sh.at[pl.ds(base + n_full * WCH, rem)], sem).wait()

    @pl.when(is_last)
    def _():
        pltpu.make_async_copy(zbuf.at[pl.ds(0, TAIL_ROWS)],
                              sh.at[pl.ds(NSUB * ROWS_PT, TAIL_ROWS)],
                              sem).wait()


def _write_cnt(sh, out_ref, stage, base, is_last):
    n_full = ROWS_PT // WCH
    rem = ROWS_PT - n_full * WCH
    for k in range(n_full):
        pltpu.sync_copy(sh.at[pl.ds(base + k * WCH, WCH)], stage)
        pltpu.sync_copy(stage, out_ref.at[pl.ds(base + k * WCH, WCH)])
    if rem:
        pltpu.sync_copy(sh.at[pl.ds(base + n_full * WCH, rem)],
                        stage.at[pl.ds(0, rem)])
        pltpu.sync_copy(stage.at[pl.ds(0, rem)],
                        out_ref.at[pl.ds(base + n_full * WCH, rem)])

    @pl.when(is_last)
    def _():
        pltpu.sync_copy(sh.at[pl.ds(NSUB * ROWS_PT, TAIL_ROWS)],
                        stage.at[pl.ds(0, TAIL_ROWS)])
        pltpu.sync_copy(stage.at[pl.ds(0, TAIL_ROWS)],
                        out_ref.at[pl.ds(NSUB * ROWS_PT, TAIL_ROWS)])


def _write_slice(sh, out_ref, stages, wsem0, wsem1, base, is_last):
    wsems = (wsem0, wsem1)
    n_full = ROWS_PT // WCH
    rem = ROWS_PT - n_full * WCH
    for k in range(n_full):
        st, sem = stages[k % 2], wsems[k % 2]
        if k >= 2:
            pltpu.make_async_copy(
                st, out_ref.at[pl.ds(base + (k - 2) * WCH, WCH)], sem).wait()
        pltpu.sync_copy(sh.at[pl.ds(base + k * WCH, WCH)], st)
        pltpu.async_copy(st, out_ref.at[pl.ds(base + k * WCH, WCH)], sem)
    for k in (n_full - 2, n_full - 1):
        st, sem = stages[k % 2], wsems[k % 2]
        pltpu.make_async_copy(
            st, out_ref.at[pl.ds(base + k * WCH, WCH)], sem).wait()
    if rem:
        st = stages[0]
        pltpu.sync_copy(sh.at[pl.ds(base + n_full * WCH, rem)],
                        st.at[pl.ds(0, rem)])
        pltpu.sync_copy(st.at[pl.ds(0, rem)],
                        out_ref.at[pl.ds(base + n_full * WCH, rem)])

    @pl.when(is_last)
    def _():
        st = stages[1]
        pltpu.sync_copy(sh.at[pl.ds(NSUB * ROWS_PT, TAIL_ROWS)],
                        st.at[pl.ds(0, TAIL_ROWS)])
        pltpu.sync_copy(st.at[pl.ds(0, TAIL_ROWS)],
                        out_ref.at[pl.ds(NSUB * ROWS_PT, TAIL_ROWS)])


NBUF = 4


def _sc_agg_body(with_cnt, hst_hbm, srcr_hbm, dstr_hbm, *rest):
    if with_cnt:
        (part_hbm, cntp_hbm, src_v, dst_v, r0, r1, r2, r3, stage, stage2,
         zbuf, ones_v, czbuf, s0, s1, s2, s3, w0, w1, w2,
         csem, isem, agg_sh, cnt_sh) = rest
    else:
        (part_hbm, src_v, dst_v, r0, r1, r2, r3, stage, stage2, zbuf,
         s0, s1, s2, s3, w0, w1, w2, csem, isem, agg_sh) = rest
    rows = (r0, r1, r2, r3)
    sems = (s0, s1, s2, s3)

    c = lax.axis_index("c")
    s = lax.axis_index("s")
    base = s * ROWS_PT
    is_last = s == NSUB - 1
    h_hbm = hst_hbm.at[c]

    pltpu.async_copy(srcr_hbm.at[s, 0], src_v, isem)
    pltpu.async_copy(dstr_hbm.at[s, 0], dst_v, isem)

    _fill_const(zbuf, WCH, HALF, 0.0)
    if with_cnt:
        _fill_const(czbuf, WCH, CNT_W, 0.0)
        _fill_const(ones_v, CHUNK, CNT_W, 1.0)

    _zero_slice(zbuf, agg_sh, base, is_last, csem)
    if with_cnt:
        _zero_slice(czbuf, cnt_sh, base, is_last, w2)

    pltpu.make_async_copy(srcr_hbm.at[s, 0], src_v, isem).wait()
    pltpu.make_async_copy(dstr_hbm.at[s, 0], dst_v, isem).wait()
    for b in range(NBUF):
        pltpu.async_copy(h_hbm.at[src_v.at[b]], rows[b], sems[b])
    plsc.subcore_barrier()

    for seg in range(NSEG):
        def group(i, _):
            j0 = NBUF * i
            for b in range(NBUF):
                j = j0 + b
                pltpu.make_async_copy(h_hbm.at[src_v.at[j]], rows[b],
                                      sems[b]).wait()
                pltpu.sync_copy(rows[b], agg_sh.at[dst_v.at[j]], add=True)
                if with_cnt:
                    @pl.when(c == seg)
                    def _():
                        pltpu.sync_copy(ones_v, cnt_sh.at[dst_v.at[j]],
                                        add=True)

                @pl.when(j + NBUF < SEG)
                def _():
                    pltpu.async_copy(h_hbm.at[src_v.at[j + NBUF]], rows[b],
                                     sems[b])
            return 0
        lax.fori_loop(0, SEG // NBUF, group, 0)

        if seg + 1 < NSEG:
            pltpu.async_copy(srcr_hbm.at[s, seg + 1], src_v, isem)
            pltpu.async_copy(dstr_hbm.at[s, seg + 1], dst_v, isem)
            pltpu.make_async_copy(srcr_hbm.at[s, seg + 1], src_v, isem).wait()
            pltpu.make_async_copy(dstr_hbm.at[s, seg + 1], dst_v, isem).wait()
            for b in range(NBUF):
                pltpu.async_copy(h_hbm.at[src_v.at[b]], rows[b], sems[b])

    plsc.subcore_barrier()

    _write_slice(agg_sh, part_hbm.at[c], (stage, stage2), w0, w1,
                 base, is_last)
    if with_cnt:
        _write_cnt(cnt_sh, cntp_hbm.at[c], czbuf, base, is_last)


def _make_sc_agg(with_cnt):
    mesh = plsc.VectorSubcoreMesh(core_axis_name="c", subcore_axis_name="s")
    out_type = [
        jax.ShapeDtypeStruct((NCORES, N_NODES, HALF), jnp.float32)]
    scratch = [
        pltpu.VMEM((SEG, CHUNK), jnp.int32),
        pltpu.VMEM((SEG, CHUNK), jnp.int32),
        pltpu.VMEM((CHUNK, HALF), jnp.float32),
        pltpu.VMEM((CHUNK, HALF), jnp.float32),
        pltpu.VMEM((CHUNK, HALF), jnp.float32),
        pltpu.VMEM((CHUNK, HALF), jnp.float32),
        pltpu.VMEM((WCH, HALF), jnp.float32),
        pltpu.VMEM((WCH, HALF), jnp.float32),
        pltpu.VMEM((WCH, HALF), jnp.float32),
        pltpu.SemaphoreType.DMA,
        pltpu.SemaphoreType.DMA,
        pltpu.SemaphoreType.DMA,
        pltpu.SemaphoreType.DMA,
        pltpu.SemaphoreType.DMA,
        pltpu.SemaphoreType.DMA,
        pltpu.SemaphoreType.DMA,
        pltpu.SemaphoreType.DMA,
        pltpu.SemaphoreType.DMA,
        pltpu.VMEM_SHARED((N_NODES, HALF), jnp.float32),
    ]
    if with_cnt:
        out_type.append(
            jax.ShapeDtypeStruct((NCORES, N_NODES, CNT_W), jnp.float32))
        scratch = scratch[:9] + [
            pltpu.VMEM((CHUNK, CNT_W), jnp.float32),
            pltpu.VMEM((WCH, CNT_W), jnp.float32),
        ] + scratch[9:] + [
            pltpu.VMEM_SHARED((N_NODES, CNT_W), jnp.float32),
        ]
    return pl.kernel(
        functools.partial(_sc_agg_body, with_cnt),
        out_type=out_type,
        mesh=mesh,
        scratch_types=scratch,
        compiler_params=pltpu.CompilerParams(use_tc_tiling_on_sc=False),
    )


def _combine_body(first, hst_ref, part_ref, aux_ref, wl_ref, bl_ref,
                  wr_ref, o_ref, *maybe_invc_out):
    h = jnp.concatenate([hst_ref[0], hst_ref[1]], axis=1)
    agg = jnp.concatenate([part_ref[0], part_ref[1]], axis=1)
    if first:
        cnt = (aux_ref[0] + aux_ref[1])[:, 0:1]
        invc = 1.0 / jnp.maximum(cnt, 1.0)
        maybe_invc_out[0][...] = invc
    else:
        invc = aux_ref[...]
    agg = agg * invc
    out = (jnp.dot(agg, wl_ref[...], preferred_element_type=jnp.float32)
           + jnp.dot(h, wr_ref[...], preferred_element_type=jnp.float32)
           + bl_ref[...])
    out = jnp.maximum(out, 0.0)
    o_ref[0] = out[:, :HALF]
    o_ref[1] = out[:, HALF:]


def _combine_tc(first, h_st, part, aux, Wl, bl, Wr):
    blk = 2000
    grid = N_NODES // blk
    if first:
        aux_spec = pl.BlockSpec((NCORES, blk, CNT_W), lambda i: (0, i, 0))
    else:
        aux_spec = pl.BlockSpec((blk, 1), lambda i: (i, 0))
    out_shape = [jax.ShapeDtypeStruct((NCORES, N_NODES, HALF), jnp.float32)]
    out_specs = [pl.BlockSpec((NCORES, blk, HALF), lambda i: (0, i, 0))]
    if first:
        out_shape.append(jax.ShapeDtypeStruct((N_NODES, 1), jnp.float32))
        out_specs.append(pl.BlockSpec((blk, 1), lambda i: (i, 0)))
    return pl.pallas_call(
        functools.partial(_combine_body, first),
        grid=(grid,),
        in_specs=[
            pl.BlockSpec((NCORES, blk, HALF), lambda i: (0, i, 0)),
            pl.BlockSpec((NCORES, blk, HALF), lambda i: (0, i, 0)),
            aux_spec,
            pl.BlockSpec((DIM, DIM), lambda i: (0, 0)),
            pl.BlockSpec((1, DIM), lambda i: (0, 0)),
            pl.BlockSpec((DIM, DIM), lambda i: (0, 0)),
        ],
        out_specs=out_specs,
        out_shape=out_shape,
    )(h_st, part, aux, Wl, bl.reshape(1, DIM), Wr)


def _last_body(nblk, hst_ref, part_ref, invc_ref, wl_ref, bl_ref,
               wr_ref, batch_ref, wc1_ref, bc1_ref, wc2_ref, bc2_ref, out_ref,
               gsum, cntg):
    i = pl.program_id(0)
    blk = hst_ref.shape[1]

    @pl.when(i == 0)
    def _():
        gsum[...] = jnp.zeros_like(gsum)
        cntg[...] = jnp.zeros_like(cntg)

    h = jnp.concatenate([hst_ref[0], hst_ref[1]], axis=1)
    agg = jnp.concatenate([part_ref[0], part_ref[1]], axis=1)
    agg = agg * invc_ref[...]
    h3 = (jnp.dot(agg, wl_ref[...], preferred_element_type=jnp.float32)
          + jnp.dot(h, wr_ref[...], preferred_element_type=jnp.float32)
          + bl_ref[...])
    h3 = jnp.maximum(h3, 0.0)

    b = batch_ref[...]
    gid = lax.broadcasted_iota(jnp.int32, (blk, N_GRAPHS), 1)
    oh = (b == gid).astype(jnp.float32)
    gsum[...] += lax.dot_general(oh, h3, (((0,), (0,)), ((), ())),
                                 preferred_element_type=jnp.float32,
                                 precision=lax.Precision.HIGHEST)
    cntg[...] += jnp.sum(oh, axis=0)[:, None]

    @pl.when(i == nblk - 1)
    def _():
        g = gsum[...] / jnp.maximum(cntg[...], 1.0)
        z = jnp.maximum(
            jnp.dot(g, wc1_ref[...], preferred_element_type=jnp.float32)
            + bc1_ref[...], 0.0)
        out_ref[...] = (
            jnp.dot(z, wc2_ref[...], preferred_element_type=jnp.float32)
            + bc2_ref[...])


def _last_tc(h_st, part, invc, Wl, bl, Wr, batch, Wc1, bc1, Wc2, bc2):
    blk = 2000
    grid = N_NODES // blk
    return pl.pallas_call(
        functools.partial(_last_body, grid),
        grid=(grid,),
        in_specs=[
            pl.BlockSpec((NCORES, blk, HALF), lambda i: (0, i, 0)),
            pl.BlockSpec((NCORES, blk, HALF), lambda i: (0, i, 0)),
            pl.BlockSpec((blk, 1), lambda i: (i, 0)),
            pl.BlockSpec((DIM, DIM), lambda i: (0, 0)),
            pl.BlockSpec((1, DIM), lambda i: (0, 0)),
            pl.BlockSpec((DIM, DIM), lambda i: (0, 0)),
            pl.BlockSpec((blk, 1), lambda i: (i, 0)),
            pl.BlockSpec((DIM, DIM), lambda i: (0, 0)),
            pl.BlockSpec((1, DIM), lambda i: (0, 0)),
            pl.BlockSpec((DIM, 2), lambda i: (0, 0)),
            pl.BlockSpec((1, 2), lambda i: (0, 0)),
        ],
        out_specs=pl.BlockSpec((N_GRAPHS, 2), lambda i: (0, 0)),
        out_shape=jax.ShapeDtypeStruct((N_GRAPHS, 2), jnp.float32),
        scratch_shapes=[pltpu.VMEM((N_GRAPHS, DIM), jnp.float32),
                        pltpu.VMEM((N_GRAPHS, 1), jnp.float32)],
    )(h_st, part, invc, Wl, bl.reshape(1, DIM), Wr,
      batch.reshape(N_NODES, 1), Wc1, bc1.reshape(1, DIM), Wc2,
      bc2.reshape(1, 2))


def kernel(x, edge_index, batch, Wl0, bl0, Wr0, Wl1, bl1, Wr1, Wl2, bl2, Wr2,
           Wc1, bc1, Wc2, bc2):
    src = edge_index[0].reshape(NSUB, NSEG, SEG, CHUNK)
    dst = edge_index[1].reshape(NSUB, NSEG, SEG, CHUNK)
    x_st = jnp.stack([x[:, :HALF], x[:, HALF:]])

    agg_cnt = _make_sc_agg(True)
    agg = _make_sc_agg(False)

    part0, cntp = agg_cnt(x_st, src, dst)
    h1, invc = _combine_tc(True, x_st, part0, cntp, Wl0, bl0, Wr0)
    part1 = agg(h1, src, dst)[0]
    h2 = _combine_tc(False, h1, part1, invc, Wl1, bl1, Wr1)[0]
    part2 = agg(h2, src, dst)[0]
    return _last_tc(h2, part2, invc, Wl2, bl2, Wr2, batch,
                    Wc1, bc1, Wc2, bc2)
